# baseline (speedup 1.0000x reference)
import functools
import math

import jax
import jax.numpy as jnp
from jax.experimental import pallas as pl
from jax.experimental.pallas import tpu as pltpu

BLOCK = 128
NEG_INF = -1e30


def _mask_body(q_ref, k_ref, ids_ref, *, nb, block, scale, min_retain,
               max_retain):
    d = q_ref.shape[-1]
    qh = q_ref[0]
    kh = k_ref[0]
    qp = qh.reshape(nb, block, d).mean(axis=1)
    kp = kh.reshape(nb, block, d).mean(axis=1)
    s = jax.lax.dot_general(qp, kp, (((1,), (1,)), ((), ())),
                            preferred_element_type=jnp.float32) * scale
    m = jnp.max(s, axis=-1, keepdims=True)
    e = jnp.exp(s - m)
    p = e / jnp.sum(e, axis=-1, keepdims=True)

    col_ids = jax.lax.broadcasted_iota(jnp.int32, (nb, nb), 1)
    colf = col_ids.astype(jnp.float32)
    rank = jnp.zeros((nb, nb), jnp.float32)
    for j in range(nb):
        col = p[:, j:j + 1]
        gt = jnp.sum((p > col).astype(jnp.float32), axis=-1, keepdims=True)
        if j > 0:
            eq = jnp.sum((p[:, :j] == col).astype(jnp.float32), axis=-1,
                         keepdims=True)
        else:
            eq = jnp.zeros_like(gt)
        rank = rank + (gt + eq) * (col_ids == j).astype(jnp.float32)

    cum = jnp.zeros((nb, nb), jnp.float32)
    for i in range(nb):
        le = (rank <= float(i)).astype(jnp.float32)
        ci = jnp.sum(p * le, axis=-1, keepdims=True)
        cum = cum + ci * (col_ids == i).astype(jnp.float32)

    thr = 0.95 * cum[:, nb - 1:nb]
    kcnt = jnp.sum((cum < thr).astype(jnp.float32), axis=-1, keepdims=True)
    kk = jnp.clip(kcnt, float(min_retain), float(max_retain))
    kept = rank < kk
    keptf = kept.astype(jnp.float32)

    row_lt_col = (jax.lax.broadcasted_iota(jnp.int32, (nb, nb), 0)
                  < col_ids).astype(jnp.float32)
    prefix = jax.lax.dot_general(keptf, row_lt_col,
                                 (((1,), (0,)), ((), ())),
                                 preferred_element_type=jnp.float32)
    last_id = jnp.max(jnp.where(kept, colf, -1.0), axis=-1, keepdims=True)

    out_cols = jax.lax.broadcasted_iota(jnp.int32, (nb, max_retain), 1)
    ids = jnp.zeros((nb, max_retain), jnp.float32)
    for pp in range(max_retain):
        sel = keptf * (prefix == float(pp)).astype(jnp.float32)
        idp = jnp.sum(colf * sel, axis=-1, keepdims=True)
        has = jnp.sum(sel, axis=-1, keepdims=True)
        idp = jnp.where(has > 0.0, idp, last_id)
        ids = ids + idp * (out_cols == pp).astype(jnp.float32)
    ids_ref[0] = ids.astype(jnp.int32)


def _attn_body(ids_smem, q_ref, k_ref, v_ref, out_ref, acc_scr, m_scr, l_scr,
               *, nb, mr, scale):
    h = pl.program_id(0)
    i = pl.program_id(1)
    j = pl.program_id(2)
    base = (h * nb + i) * mr

    @pl.when(j == 0)
    def _():
        acc_scr[...] = jnp.zeros(acc_scr.shape, jnp.float32)
        m_scr[...] = jnp.full(m_scr.shape, NEG_INF, jnp.float32)
        l_scr[...] = jnp.zeros(l_scr.shape, jnp.float32)

    idc = ids_smem[base + j]
    idp = ids_smem[jnp.maximum(base + j - 1, 0)]
    valid = jnp.logical_or(j == 0, idc != idp)

    @pl.when(valid)
    def _():
        qb = q_ref[0]
        kj = k_ref[0]
        s = jax.lax.dot_general(qb, kj, (((1,), (1,)), ((), ())),
                                preferred_element_type=jnp.float32) * scale
        m_prev = m_scr[...]
        m_cur = jnp.max(s, axis=-1, keepdims=True)
        m_next = jnp.maximum(m_prev, m_cur)
        alpha = jnp.exp(m_prev - m_next)
        pmat = jnp.exp(s - m_next)
        l_scr[...] = l_scr[...] * alpha + jnp.sum(pmat, axis=-1, keepdims=True)
        acc_scr[...] = acc_scr[...] * alpha + jax.lax.dot_general(
            pmat, v_ref[0], (((1,), (0,)), ((), ())),
            preferred_element_type=jnp.float32)
        m_scr[...] = m_next

    @pl.when(j == mr - 1)
    def _():
        out_ref[0] = acc_scr[...] / l_scr[...]


@jax.jit
def kernel(q, k, v):
    B, H, S, d = q.shape
    nb = S // BLOCK
    BH = B * H
    scale = 1.0 / math.sqrt(d)
    min_retain = max(1, int(nb * 0.05))
    max_retain = max(1, int(nb * 0.7))
    mr = max_retain

    qf = q.reshape(BH, S, d)
    kf = k.reshape(BH, S, d)
    vf = v.reshape(BH, S, d)

    ids = pl.pallas_call(
        functools.partial(_mask_body, nb=nb, block=BLOCK, scale=scale,
                          min_retain=min_retain, max_retain=max_retain),
        grid=(BH,),
        in_specs=[
            pl.BlockSpec((1, S, d), lambda h: (h, 0, 0)),
            pl.BlockSpec((1, S, d), lambda h: (h, 0, 0)),
        ],
        out_specs=pl.BlockSpec((1, nb, mr), lambda h: (h, 0, 0)),
        out_shape=jax.ShapeDtypeStruct((BH, nb, mr), jnp.int32),
        compiler_params=pltpu.CompilerParams(
            dimension_semantics=("arbitrary",)),
    )(qf, kf)

    ids_flat = ids.reshape(-1)

    grid_spec = pltpu.PrefetchScalarGridSpec(
        num_scalar_prefetch=1,
        grid=(BH, nb, mr),
        in_specs=[
            pl.BlockSpec((1, BLOCK, d), lambda h, i, j, ids: (h, i, 0)),
            pl.BlockSpec((1, BLOCK, d),
                         lambda h, i, j, ids: (h, ids[(h * nb + i) * mr + j],
                                               0)),
            pl.BlockSpec((1, BLOCK, d),
                         lambda h, i, j, ids: (h, ids[(h * nb + i) * mr + j],
                                               0)),
        ],
        out_specs=pl.BlockSpec((1, BLOCK, d), lambda h, i, j, ids: (h, i, 0)),
        scratch_shapes=[
            pltpu.VMEM((BLOCK, d), jnp.float32),
            pltpu.VMEM((BLOCK, BLOCK), jnp.float32),
            pltpu.VMEM((BLOCK, BLOCK), jnp.float32),
        ],
    )
    out = pl.pallas_call(
        functools.partial(_attn_body, nb=nb, mr=mr, scale=scale),
        grid_spec=grid_spec,
        out_shape=jax.ShapeDtypeStruct((BH, S, d), jnp.float32),
        compiler_params=pltpu.CompilerParams(
            dimension_semantics=("parallel", "parallel", "arbitrary")),
    )(ids_flat, qf, kf, vf)

    return out.reshape(B, H, S, d)


# speedup vs baseline: 2.5436x; 2.5436x over previous
import functools
import math

import jax
import jax.numpy as jnp
from jax.experimental import pallas as pl
from jax.experimental.pallas import tpu as pltpu

BLOCK = 128
NEG_INF = -1e30


def _mask_body(q_ref, k_ref, ids_ref, *, nb, block, scale, min_retain,
               max_retain):
    d = q_ref.shape[-1]
    qh = q_ref[0]
    kh = k_ref[0]
    qp = qh.reshape(nb, block, d).mean(axis=1)
    kp = kh.reshape(nb, block, d).mean(axis=1)
    s = jax.lax.dot_general(qp, kp, (((1,), (1,)), ((), ())),
                            preferred_element_type=jnp.float32) * scale
    m = jnp.max(s, axis=-1, keepdims=True)
    e = jnp.exp(s - m)
    p = e / jnp.sum(e, axis=-1, keepdims=True)

    col_ids = jax.lax.broadcasted_iota(jnp.int32, (nb, nb), 1)
    colf = col_ids.astype(jnp.float32)
    rank = jnp.zeros((nb, nb), jnp.float32)
    for j in range(nb):
        col = p[:, j:j + 1]
        gt = jnp.sum((p > col).astype(jnp.float32), axis=-1, keepdims=True)
        if j > 0:
            eq = jnp.sum((p[:, :j] == col).astype(jnp.float32), axis=-1,
                         keepdims=True)
        else:
            eq = jnp.zeros_like(gt)
        rank = rank + (gt + eq) * (col_ids == j).astype(jnp.float32)

    cum = jnp.zeros((nb, nb), jnp.float32)
    for i in range(nb):
        le = (rank <= float(i)).astype(jnp.float32)
        ci = jnp.sum(p * le, axis=-1, keepdims=True)
        cum = cum + ci * (col_ids == i).astype(jnp.float32)

    thr = 0.95 * cum[:, nb - 1:nb]
    kcnt = jnp.sum((cum < thr).astype(jnp.float32), axis=-1, keepdims=True)
    kk = jnp.clip(kcnt, float(min_retain), float(max_retain))
    kept = rank < kk
    keptf = kept.astype(jnp.float32)

    row_lt_col = (jax.lax.broadcasted_iota(jnp.int32, (nb, nb), 0)
                  < col_ids).astype(jnp.float32)
    prefix = jax.lax.dot_general(keptf, row_lt_col,
                                 (((1,), (0,)), ((), ())),
                                 preferred_element_type=jnp.float32)
    last_id = jnp.max(jnp.where(kept, colf, -1.0), axis=-1, keepdims=True)

    out_cols = jax.lax.broadcasted_iota(jnp.int32, (nb, max_retain), 1)
    ids = jnp.zeros((nb, max_retain), jnp.float32)
    for pp in range(max_retain):
        sel = keptf * (prefix == float(pp)).astype(jnp.float32)
        idp = jnp.sum(colf * sel, axis=-1, keepdims=True)
        has = jnp.sum(sel, axis=-1, keepdims=True)
        idp = jnp.where(has > 0.0, idp, last_id)
        ids = ids + idp * (out_cols == pp).astype(jnp.float32)
    ids_ref[0] = ids.astype(jnp.int32)


def _attn_body(ids_smem, q_ref, k_ref, v_ref, out_ref, *, nb, mr, block,
               scale):
    h = pl.program_id(0)
    i = pl.program_id(1)
    base = (h * nb + i) * mr

    qb = q_ref[0]

    id0 = ids_smem[base]
    k0 = k_ref[0, pl.ds(id0 * block, block), :]
    s0 = jax.lax.dot_general(qb, k0, (((1,), (1,)), ((), ())),
                             preferred_element_type=jnp.float32) * scale
    m = jnp.max(s0, axis=-1, keepdims=True)
    p0 = jnp.exp(s0 - m)
    l = jnp.sum(p0, axis=-1, keepdims=True)
    v0 = v_ref[0, pl.ds(id0 * block, block), :]
    acc = jax.lax.dot_general(p0, v0, (((1,), (0,)), ((), ())),
                              preferred_element_type=jnp.float32)

    for j in range(1, mr):
        idj = ids_smem[base + j]
        idp = ids_smem[base + j - 1]
        valid = idj != idp
        kj = k_ref[0, pl.ds(idj * block, block), :]
        s = jax.lax.dot_general(qb, kj, (((1,), (1,)), ((), ())),
                                preferred_element_type=jnp.float32) * scale
        s = jnp.where(valid, s, NEG_INF)
        m_cur = jnp.max(s, axis=-1, keepdims=True)
        m_new = jnp.maximum(m, m_cur)
        alpha = jnp.exp(m - m_new)
        pmat = jnp.exp(s - m_new)
        l = l * alpha + jnp.sum(pmat, axis=-1, keepdims=True)
        vj = v_ref[0, pl.ds(idj * block, block), :]
        acc = acc * alpha + jax.lax.dot_general(
            pmat, vj, (((1,), (0,)), ((), ())),
            preferred_element_type=jnp.float32)
        m = m_new

    out_ref[0] = acc / l


@jax.jit
def kernel(q, k, v):
    B, H, S, d = q.shape
    nb = S // BLOCK
    BH = B * H
    scale = 1.0 / math.sqrt(d)
    min_retain = max(1, int(nb * 0.05))
    max_retain = max(1, int(nb * 0.7))
    mr = max_retain

    qf = q.reshape(BH, S, d)
    kf = k.reshape(BH, S, d)
    vf = v.reshape(BH, S, d)

    ids = pl.pallas_call(
        functools.partial(_mask_body, nb=nb, block=BLOCK, scale=scale,
                          min_retain=min_retain, max_retain=max_retain),
        grid=(BH,),
        in_specs=[
            pl.BlockSpec((1, S, d), lambda h: (h, 0, 0)),
            pl.BlockSpec((1, S, d), lambda h: (h, 0, 0)),
        ],
        out_specs=pl.BlockSpec((1, nb, mr), lambda h: (h, 0, 0)),
        out_shape=jax.ShapeDtypeStruct((BH, nb, mr), jnp.int32),
        compiler_params=pltpu.CompilerParams(
            dimension_semantics=("arbitrary",)),
    )(qf, kf)

    ids_flat = ids.reshape(-1)

    grid_spec = pltpu.PrefetchScalarGridSpec(
        num_scalar_prefetch=1,
        grid=(BH, nb),
        in_specs=[
            pl.BlockSpec((1, BLOCK, d), lambda h, i, ids: (h, i, 0)),
            pl.BlockSpec((1, S, d), lambda h, i, ids: (h, 0, 0)),
            pl.BlockSpec((1, S, d), lambda h, i, ids: (h, 0, 0)),
        ],
        out_specs=pl.BlockSpec((1, BLOCK, d), lambda h, i, ids: (h, i, 0)),
    )
    out = pl.pallas_call(
        functools.partial(_attn_body, nb=nb, mr=mr, block=BLOCK, scale=scale),
        grid_spec=grid_spec,
        out_shape=jax.ShapeDtypeStruct((BH, S, d), jnp.float32),
        compiler_params=pltpu.CompilerParams(
            dimension_semantics=("parallel", "arbitrary")),
    )(ids_flat, qf, kf, vf)

    return out.reshape(B, H, S, d)


# speedup vs baseline: 6.3910x; 2.5126x over previous
import functools
import math

import jax
import jax.numpy as jnp
from jax.experimental import pallas as pl
from jax.experimental.pallas import tpu as pltpu

BLOCK = 128
NEG_INF = -1e30


def _mask_body(q_ref, k_ref, ids_ref, *, nb, block, scale, min_retain,
               max_retain):
    d = q_ref.shape[-1]
    qh = q_ref[0]
    kh = k_ref[0]
    qp = qh.reshape(nb, block, d).mean(axis=1)
    kp = kh.reshape(nb, block, d).mean(axis=1)
    s = jax.lax.dot_general(qp, kp, (((1,), (1,)), ((), ())),
                            preferred_element_type=jnp.float32) * scale
    m = jnp.max(s, axis=-1, keepdims=True)
    e = jnp.exp(s - m)
    p = e / jnp.sum(e, axis=-1, keepdims=True)

    col_ids = jax.lax.broadcasted_iota(jnp.int32, (nb, nb), 1)
    colf = col_ids.astype(jnp.float32)
    rank = jnp.zeros((nb, nb), jnp.float32)
    for j in range(nb):
        col = p[:, j:j + 1]
        gt = jnp.sum((p > col).astype(jnp.float32), axis=-1, keepdims=True)
        if j > 0:
            eq = jnp.sum((p[:, :j] == col).astype(jnp.float32), axis=-1,
                         keepdims=True)
        else:
            eq = jnp.zeros_like(gt)
        rank = rank + (gt + eq) * (col_ids == j).astype(jnp.float32)

    cum = jnp.zeros((nb, nb), jnp.float32)
    for i in range(nb):
        le = (rank <= float(i)).astype(jnp.float32)
        ci = jnp.sum(p * le, axis=-1, keepdims=True)
        cum = cum + ci * (col_ids == i).astype(jnp.float32)

    thr = 0.95 * cum[:, nb - 1:nb]
    kcnt = jnp.sum((cum < thr).astype(jnp.float32), axis=-1, keepdims=True)
    kk = jnp.clip(kcnt, float(min_retain), float(max_retain))
    kept = rank < kk
    keptf = kept.astype(jnp.float32)

    row_lt_col = (jax.lax.broadcasted_iota(jnp.int32, (nb, nb), 0)
                  < col_ids).astype(jnp.float32)
    prefix = jax.lax.dot_general(keptf, row_lt_col,
                                 (((1,), (0,)), ((), ())),
                                 preferred_element_type=jnp.float32)
    last_id = jnp.max(jnp.where(kept, colf, -1.0), axis=-1, keepdims=True)

    out_cols = jax.lax.broadcasted_iota(jnp.int32, (nb, max_retain), 1)
    ids = jnp.zeros((nb, max_retain), jnp.float32)
    for pp in range(max_retain):
        sel = keptf * (prefix == float(pp)).astype(jnp.float32)
        idp = jnp.sum(colf * sel, axis=-1, keepdims=True)
        has = jnp.sum(sel, axis=-1, keepdims=True)
        idp = jnp.where(has > 0.0, idp, last_id)
        ids = ids + idp * (out_cols == pp).astype(jnp.float32)
    ids_ref[0] = ids.astype(jnp.int32)


def _attn_body(ids_smem, q_ref, k_ref, v_ref, out_ref, *, nb, mr, block,
               scale):
    h = pl.program_id(0)
    i = pl.program_id(1)
    base = (h * nb + i) * mr

    qb = q_ref[0] * scale

    s_blocks = []
    for j in range(mr):
        idj = ids_smem[base + j]
        kj = k_ref[0, pl.ds(idj * block, block), :]
        s = jax.lax.dot_general(qb, kj, (((1,), (1,)), ((), ())),
                                preferred_element_type=jnp.float32)
        if j > 0:
            valid = idj != ids_smem[base + j - 1]
            s = jnp.where(valid, s, NEG_INF)
        s_blocks.append(s)

    m = s_blocks[0]
    for j in range(1, mr):
        m = jnp.maximum(m, s_blocks[j])
    m = jnp.max(m, axis=-1, keepdims=True)
    p_blocks = [jnp.exp(s - m) for s in s_blocks]

    l = p_blocks[0]
    for j in range(1, mr):
        l = l + p_blocks[j]
    l = jnp.sum(l, axis=-1, keepdims=True)

    outs = []
    for j in range(mr):
        idj = ids_smem[base + j]
        vj = v_ref[0, pl.ds(idj * block, block), :]
        outs.append(jax.lax.dot_general(p_blocks[j], vj,
                                        (((1,), (0,)), ((), ())),
                                        preferred_element_type=jnp.float32))
    while len(outs) > 1:
        outs = [outs[a] + outs[a + 1] if a + 1 < len(outs) else outs[a]
                for a in range(0, len(outs), 2)]

    out_ref[0] = outs[0] / l


@jax.jit
def kernel(q, k, v):
    B, H, S, d = q.shape
    nb = S // BLOCK
    BH = B * H
    scale = 1.0 / math.sqrt(d)
    min_retain = max(1, int(nb * 0.05))
    max_retain = max(1, int(nb * 0.7))
    mr = max_retain

    qf = q.reshape(BH, S, d)
    kf = k.reshape(BH, S, d)
    vf = v.reshape(BH, S, d)

    ids = pl.pallas_call(
        functools.partial(_mask_body, nb=nb, block=BLOCK, scale=scale,
                          min_retain=min_retain, max_retain=max_retain),
        grid=(BH,),
        in_specs=[
            pl.BlockSpec((1, S, d), lambda h: (h, 0, 0)),
            pl.BlockSpec((1, S, d), lambda h: (h, 0, 0)),
        ],
        out_specs=pl.BlockSpec((1, nb, mr), lambda h: (h, 0, 0)),
        out_shape=jax.ShapeDtypeStruct((BH, nb, mr), jnp.int32),
        compiler_params=pltpu.CompilerParams(
            dimension_semantics=("arbitrary",)),
    )(qf, kf)

    ids_flat = ids.reshape(-1)

    grid_spec = pltpu.PrefetchScalarGridSpec(
        num_scalar_prefetch=1,
        grid=(BH, nb),
        in_specs=[
            pl.BlockSpec((1, BLOCK, d), lambda h, i, ids: (h, i, 0)),
            pl.BlockSpec((1, S, d), lambda h, i, ids: (h, 0, 0)),
            pl.BlockSpec((1, S, d), lambda h, i, ids: (h, 0, 0)),
        ],
        out_specs=pl.BlockSpec((1, BLOCK, d), lambda h, i, ids: (h, i, 0)),
    )
    out = pl.pallas_call(
        functools.partial(_attn_body, nb=nb, mr=mr, block=BLOCK, scale=scale),
        grid_spec=grid_spec,
        out_shape=jax.ShapeDtypeStruct((BH, S, d), jnp.float32),
        compiler_params=pltpu.CompilerParams(
            dimension_semantics=("parallel", "arbitrary")),
    )(ids_flat, qf, kf, vf)

    return out.reshape(B, H, S, d)


# speedup vs baseline: 6.4144x; 1.0037x over previous
import functools
import math

import jax
import jax.numpy as jnp
from jax.experimental import pallas as pl
from jax.experimental.pallas import tpu as pltpu

BLOCK = 128
NEG_INF = -1e30


def _mask_body(q_ref, k_ref, ids_ref, *, nb, block, scale, min_retain,
               max_retain):
    d = q_ref.shape[-1]
    qh = q_ref[0]
    kh = k_ref[0]
    qp = qh.reshape(nb, block, d).mean(axis=1)
    kp = kh.reshape(nb, block, d).mean(axis=1)
    s = jax.lax.dot_general(qp, kp, (((1,), (1,)), ((), ())),
                            preferred_element_type=jnp.float32) * scale
    m = jnp.max(s, axis=-1, keepdims=True)
    e = jnp.exp(s - m)
    p = e / jnp.sum(e, axis=-1, keepdims=True)

    col_ids = jax.lax.broadcasted_iota(jnp.int32, (nb, nb), 1)
    colf = col_ids.astype(jnp.float32)
    rank = jnp.zeros((nb, nb), jnp.float32)
    for j in range(nb):
        col = p[:, j:j + 1]
        gt = jnp.sum((p > col).astype(jnp.float32), axis=-1, keepdims=True)
        if j > 0:
            eq = jnp.sum((p[:, :j] == col).astype(jnp.float32), axis=-1,
                         keepdims=True)
        else:
            eq = jnp.zeros_like(gt)
        rank = rank + (gt + eq) * (col_ids == j).astype(jnp.float32)

    cum = jnp.zeros((nb, nb), jnp.float32)
    for i in range(nb):
        le = (rank <= float(i)).astype(jnp.float32)
        ci = jnp.sum(p * le, axis=-1, keepdims=True)
        cum = cum + ci * (col_ids == i).astype(jnp.float32)

    thr = 0.95 * cum[:, nb - 1:nb]
    kcnt = jnp.sum((cum < thr).astype(jnp.float32), axis=-1, keepdims=True)
    kk = jnp.clip(kcnt, float(min_retain), float(max_retain))
    kept = rank < kk
    keptf = kept.astype(jnp.float32)

    row_lt_col = (jax.lax.broadcasted_iota(jnp.int32, (nb, nb), 0)
                  < col_ids).astype(jnp.float32)
    prefix = jax.lax.dot_general(keptf, row_lt_col,
                                 (((1,), (0,)), ((), ())),
                                 preferred_element_type=jnp.float32)
    last_id = jnp.max(jnp.where(kept, colf, -1.0), axis=-1, keepdims=True)

    out_cols = jax.lax.broadcasted_iota(jnp.int32, (nb, max_retain), 1)
    ids = jnp.zeros((nb, max_retain), jnp.float32)
    for pp in range(max_retain):
        sel = keptf * (prefix == float(pp)).astype(jnp.float32)
        idp = jnp.sum(colf * sel, axis=-1, keepdims=True)
        has = jnp.sum(sel, axis=-1, keepdims=True)
        idp = jnp.where(has > 0.0, idp, last_id)
        ids = ids + idp * (out_cols == pp).astype(jnp.float32)
    ids_ref[0] = ids.astype(jnp.int32)


def _attn_body(ids_smem, q_ref, k_ref, v_ref, out_ref, *, nb, mr, block,
               scale):
    h = pl.program_id(0)
    i = pl.program_id(1)
    base = (h * nb + i) * mr

    qb = (q_ref[0] * scale).astype(jnp.bfloat16)

    s_blocks = []
    for j in range(mr):
        idj = ids_smem[base + j]
        kj = k_ref[0, pl.ds(idj * block, block), :].astype(jnp.bfloat16)
        s = jax.lax.dot_general(qb, kj, (((1,), (1,)), ((), ())),
                                preferred_element_type=jnp.float32)
        if j > 0:
            valid = idj != ids_smem[base + j - 1]
            s = jnp.where(valid, s, NEG_INF)
        s_blocks.append(s)

    m = s_blocks[0]
    for j in range(1, mr):
        m = jnp.maximum(m, s_blocks[j])
    m = jnp.max(m, axis=-1, keepdims=True)
    p_blocks = [jnp.exp(s - m) for s in s_blocks]

    l = p_blocks[0]
    for j in range(1, mr):
        l = l + p_blocks[j]
    l = jnp.sum(l, axis=-1, keepdims=True)

    outs = []
    for j in range(mr):
        idj = ids_smem[base + j]
        vj = v_ref[0, pl.ds(idj * block, block), :].astype(jnp.bfloat16)
        pj = p_blocks[j].astype(jnp.bfloat16)
        outs.append(jax.lax.dot_general(pj, vj,
                                        (((1,), (0,)), ((), ())),
                                        preferred_element_type=jnp.float32))
    while len(outs) > 1:
        outs = [outs[a] + outs[a + 1] if a + 1 < len(outs) else outs[a]
                for a in range(0, len(outs), 2)]

    out_ref[0] = outs[0] / l


@jax.jit
def kernel(q, k, v):
    B, H, S, d = q.shape
    nb = S // BLOCK
    BH = B * H
    scale = 1.0 / math.sqrt(d)
    min_retain = max(1, int(nb * 0.05))
    max_retain = max(1, int(nb * 0.7))
    mr = max_retain

    qf = q.reshape(BH, S, d)
    kf = k.reshape(BH, S, d)
    vf = v.reshape(BH, S, d)

    ids = pl.pallas_call(
        functools.partial(_mask_body, nb=nb, block=BLOCK, scale=scale,
                          min_retain=min_retain, max_retain=max_retain),
        grid=(BH,),
        in_specs=[
            pl.BlockSpec((1, S, d), lambda h: (h, 0, 0)),
            pl.BlockSpec((1, S, d), lambda h: (h, 0, 0)),
        ],
        out_specs=pl.BlockSpec((1, nb, mr), lambda h: (h, 0, 0)),
        out_shape=jax.ShapeDtypeStruct((BH, nb, mr), jnp.int32),
        compiler_params=pltpu.CompilerParams(
            dimension_semantics=("arbitrary",)),
    )(qf, kf)

    ids_flat = ids.reshape(-1)

    grid_spec = pltpu.PrefetchScalarGridSpec(
        num_scalar_prefetch=1,
        grid=(BH, nb),
        in_specs=[
            pl.BlockSpec((1, BLOCK, d), lambda h, i, ids: (h, i, 0)),
            pl.BlockSpec((1, S, d), lambda h, i, ids: (h, 0, 0)),
            pl.BlockSpec((1, S, d), lambda h, i, ids: (h, 0, 0)),
        ],
        out_specs=pl.BlockSpec((1, BLOCK, d), lambda h, i, ids: (h, i, 0)),
    )
    out = pl.pallas_call(
        functools.partial(_attn_body, nb=nb, mr=mr, block=BLOCK, scale=scale),
        grid_spec=grid_spec,
        out_shape=jax.ShapeDtypeStruct((BH, S, d), jnp.float32),
        compiler_params=pltpu.CompilerParams(
            dimension_semantics=("parallel", "arbitrary")),
    )(ids_flat, qf, kf, vf)

    return out.reshape(B, H, S, d)
